# baseline (speedup 1.0000x reference)
import functools

import jax
import jax.numpy as jnp
from jax.experimental import pallas as pl
from jax.experimental.pallas import tpu as pltpu
from jax.experimental.pallas import tpu_sc as plsc

H = 2048
H2 = 1024
NT = 55

_GATHER_WINDOW = 16
_TOK_BLOCK = 512
_CHUNKS = (2048, 14336)


def _sc_gather(table, ids):
    n = ids.shape[0]
    info = plsc.get_sparse_core_info()
    nw = info.num_cores * info.num_subcores
    per_w = n // nw
    c = _GATHER_WINDOW
    mesh = plsc.VectorSubcoreMesh(core_axis_name="c", subcore_axis_name="s")

    @functools.partial(
        pl.kernel,
        out_type=jax.ShapeDtypeStruct((n, H), jnp.float32),
        mesh=mesh,
        scratch_types=[
            pltpu.VMEM((per_w,), jnp.int32),
            pltpu.VMEM((c, H), jnp.float32),
            pltpu.SemaphoreType.DMA,
        ],
    )
    def gather_kernel(table_hbm, ids_hbm, out_hbm, idx_v, rows_v, sem):
        wid = jax.lax.axis_index("s") * info.num_cores + jax.lax.axis_index("c")
        base = wid * per_w
        pltpu.sync_copy(ids_hbm.at[pl.ds(base, per_w)], idx_v)

        @pl.loop(0, per_w, step=c)
        def _(i):
            pltpu.async_copy(
                table_hbm.at[idx_v.at[pl.ds(i, c)]], rows_v, sem).wait()
            pltpu.sync_copy(rows_v, out_hbm.at[pl.ds(base + i, c)])

    return gather_kernel(table, ids)


def _mlp_body(tok_ref, w1_ref, b1_ref, w2_ref, b2_ref, lng_ref, lnb_ref,
              wh1_ref, bh1_ref, wh2_ref, bh2_ref, comb_ref, log_ref,
              tokb_s, gb_s, lnb_s, rb_s):
    tok = tok_ref[...]
    tokb_s[...] = tok.astype(jnp.bfloat16)
    h1 = jnp.dot(tokb_s[...], w1_ref[...],
                 preferred_element_type=jnp.float32) + b1_ref[...]
    g = 0.5 * h1 * (1.0 + jax.lax.erf(h1 * 0.7071067811865476))
    gb_s[...] = g.astype(jnp.bfloat16)
    h = jnp.dot(gb_s[...], w2_ref[...],
                preferred_element_type=jnp.float32) + b2_ref[...]
    comb = tok + h
    comb_ref[...] = comb
    mu = jnp.mean(comb, axis=-1, keepdims=True)
    var = jnp.mean((comb - mu) ** 2, axis=-1, keepdims=True)
    ln = (comb - mu) * jax.lax.rsqrt(var + 1e-5) * lng_ref[...] + lnb_ref[...]
    lnb_s[...] = ln.astype(jnp.bfloat16)
    a = jnp.dot(lnb_s[...], wh1_ref[...],
                preferred_element_type=jnp.float32) + bh1_ref[...]
    rb_s[...] = jnp.maximum(a, 0.0).astype(jnp.bfloat16)
    log_ref[...] = jnp.dot(rb_s[...], wh2_ref[...],
                           preferred_element_type=jnp.float32) + bh2_ref[...]


def _cast_weights(W1, W2, Wh1):
    r = 256

    def body(w1_ref, w2_ref, wh1_ref, o1_ref, o2_ref, o3_ref):
        o1_ref[...] = w1_ref[...].astype(jnp.bfloat16)
        o2_ref[...] = w2_ref[...].astype(jnp.bfloat16)
        o3_ref[...] = wh1_ref[...].astype(jnp.bfloat16)

    return pl.pallas_call(
        body,
        grid=(H // r,),
        in_specs=[
            pl.BlockSpec((r, H), lambda i: (i, 0)),
            pl.BlockSpec((r, H), lambda i: (i, 0)),
            pl.BlockSpec((r, H2), lambda i: (i, 0)),
        ],
        out_specs=[
            pl.BlockSpec((r, H), lambda i: (i, 0)),
            pl.BlockSpec((r, H), lambda i: (i, 0)),
            pl.BlockSpec((r, H2), lambda i: (i, 0)),
        ],
        out_shape=[
            jax.ShapeDtypeStruct((H, H), jnp.bfloat16),
            jax.ShapeDtypeStruct((H, H), jnp.bfloat16),
            jax.ShapeDtypeStruct((H, H2), jnp.bfloat16),
        ],
    )(W1, W2, Wh1)


def _tc_mlp_chunk(tok_chunk, n_total, blk_off, weights, carry, interpret=False):
    t = _TOK_BLOCK
    g = tok_chunk.shape[0] // t
    full = lambda shape: pl.BlockSpec(shape, lambda i: (0, 0))
    in_specs = [
        pl.BlockSpec((t, H), lambda i: (i, 0)),
        full((H, H)), full((1, H)),
        full((H, H)), full((1, H)),
        full((1, H)), full((1, H)),
        full((H, H2)), full((1, H2)),
        full((H2, NT)), full((1, NT)),
    ]
    args = [tok_chunk] + list(weights)
    aliases = {}
    if carry is not None:
        in_specs += [pl.BlockSpec(memory_space=pl.ANY),
                     pl.BlockSpec(memory_space=pl.ANY)]
        args += [carry[0], carry[1]]
        aliases = {11: 0, 12: 1}

    def body(*refs):
        _mlp_body(*refs[:11], *refs[-6:])

    return pl.pallas_call(
        body,
        grid=(g,),
        in_specs=in_specs,
        scratch_shapes=[
            pltpu.VMEM((t, H), jnp.bfloat16),
            pltpu.VMEM((t, H), jnp.bfloat16),
            pltpu.VMEM((t, H), jnp.bfloat16),
            pltpu.VMEM((t, H2), jnp.bfloat16),
        ],
        out_specs=[
            pl.BlockSpec((t, H), lambda i: (blk_off + i, 0)),
            pl.BlockSpec((t, NT), lambda i: (blk_off + i, 0)),
        ],
        out_shape=[
            jax.ShapeDtypeStruct((n_total, H), jnp.float32),
            jax.ShapeDtypeStruct((n_total, NT), jnp.float32),
        ],
        input_output_aliases=aliases,
        interpret=interpret,
    )(*args)


def kernel(input_ids, table, W1, b1, W2, b2, ln_g, ln_b, Wh1, bh1, Wh2, bh2):
    b, s = input_ids.shape
    n = b * s
    ids = input_ids.reshape(-1).astype(jnp.int32)
    W1b, W2b, Wh1b = _cast_weights(W1, W2, Wh1)
    weights = (W1b, b1.reshape(1, H),
               W2b, b2.reshape(1, H),
               ln_g.reshape(1, H), ln_b.reshape(1, H),
               Wh1b, bh1.reshape(1, H2),
               Wh2.astype(jnp.bfloat16), bh2.reshape(1, NT))
    offs = [0]
    for c in _CHUNKS:
        offs.append(offs[-1] + c)
    toks = [_sc_gather(table, jax.lax.dynamic_slice_in_dim(ids, offs[k], c))
            for k, c in enumerate(_CHUNKS)]
    carry = None
    for k in range(len(_CHUNKS)):
        carry = _tc_mlp_chunk(toks[k], n, offs[k] // _TOK_BLOCK,
                              weights, carry)
    comb, logits = carry
    return comb.reshape(b, s, H), logits.reshape(b, s, NT)


# speedup vs baseline: 1.0440x; 1.0440x over previous
import functools

import jax
import jax.numpy as jnp
from jax.experimental import pallas as pl
from jax.experimental.pallas import tpu as pltpu
from jax.experimental.pallas import tpu_sc as plsc

H = 2048
H2 = 1024
NT = 55

_GATHER_WINDOW = 16
_TOK_BLOCK = 512
_CHUNKS = (3072, 13312)


def _sc_gather(table, ids):
    n = ids.shape[0]
    info = plsc.get_sparse_core_info()
    nw = info.num_cores * info.num_subcores
    per_w = n // nw
    c = _GATHER_WINDOW
    mesh = plsc.VectorSubcoreMesh(core_axis_name="c", subcore_axis_name="s")

    @functools.partial(
        pl.kernel,
        out_type=jax.ShapeDtypeStruct((n, H), jnp.float32),
        mesh=mesh,
        scratch_types=[
            pltpu.VMEM((per_w,), jnp.int32),
            pltpu.VMEM((c, H), jnp.float32),
            pltpu.SemaphoreType.DMA,
        ],
    )
    def gather_kernel(table_hbm, ids_hbm, out_hbm, idx_v, rows_v, sem):
        wid = jax.lax.axis_index("s") * info.num_cores + jax.lax.axis_index("c")
        base = wid * per_w
        pltpu.sync_copy(ids_hbm.at[pl.ds(base, per_w)], idx_v)

        @pl.loop(0, per_w, step=c)
        def _(i):
            pltpu.async_copy(
                table_hbm.at[idx_v.at[pl.ds(i, c)]], rows_v, sem).wait()
            pltpu.sync_copy(rows_v, out_hbm.at[pl.ds(base + i, c)])

    return gather_kernel(table, ids)


def _mlp_body(tok_ref, w1_ref, b1_ref, w2_ref, b2_ref, lng_ref, lnb_ref,
              wh1_ref, bh1_ref, wh2_ref, bh2_ref, comb_ref, log_ref,
              tokb_s, gb_s, lnb_s, rb_s):
    tok = tok_ref[...]
    tokb_s[...] = tok.astype(jnp.bfloat16)
    h1 = jnp.dot(tokb_s[...], w1_ref[...],
                 preferred_element_type=jnp.float32) + b1_ref[...]
    g = 0.5 * h1 * (1.0 + jax.lax.erf(h1 * 0.7071067811865476))
    gb_s[...] = g.astype(jnp.bfloat16)
    h = jnp.dot(gb_s[...], w2_ref[...],
                preferred_element_type=jnp.float32) + b2_ref[...]
    comb = tok + h
    comb_ref[...] = comb
    mu = jnp.mean(comb, axis=-1, keepdims=True)
    var = jnp.mean((comb - mu) ** 2, axis=-1, keepdims=True)
    ln = (comb - mu) * jax.lax.rsqrt(var + 1e-5) * lng_ref[...] + lnb_ref[...]
    lnb_s[...] = ln.astype(jnp.bfloat16)
    a = jnp.dot(lnb_s[...], wh1_ref[...],
                preferred_element_type=jnp.float32) + bh1_ref[...]
    rb_s[...] = jnp.maximum(a, 0.0).astype(jnp.bfloat16)
    log_ref[...] = jnp.dot(rb_s[...], wh2_ref[...],
                           preferred_element_type=jnp.float32) + bh2_ref[...]


def _cast_weights(W1, W2, Wh1):
    r = 256

    def body(w1_ref, w2_ref, wh1_ref, o1_ref, o2_ref, o3_ref):
        o1_ref[...] = w1_ref[...].astype(jnp.bfloat16)
        o2_ref[...] = w2_ref[...].astype(jnp.bfloat16)
        o3_ref[...] = wh1_ref[...].astype(jnp.bfloat16)

    return pl.pallas_call(
        body,
        grid=(H // r,),
        in_specs=[
            pl.BlockSpec((r, H), lambda i: (i, 0)),
            pl.BlockSpec((r, H), lambda i: (i, 0)),
            pl.BlockSpec((r, H2), lambda i: (i, 0)),
        ],
        out_specs=[
            pl.BlockSpec((r, H), lambda i: (i, 0)),
            pl.BlockSpec((r, H), lambda i: (i, 0)),
            pl.BlockSpec((r, H2), lambda i: (i, 0)),
        ],
        out_shape=[
            jax.ShapeDtypeStruct((H, H), jnp.bfloat16),
            jax.ShapeDtypeStruct((H, H), jnp.bfloat16),
            jax.ShapeDtypeStruct((H, H2), jnp.bfloat16),
        ],
    )(W1, W2, Wh1)


def _tc_mlp_chunk(tok_chunk, n_total, blk_off, weights, carry, interpret=False):
    t = _TOK_BLOCK
    g = tok_chunk.shape[0] // t
    full = lambda shape: pl.BlockSpec(shape, lambda i: (0, 0))
    in_specs = [
        pl.BlockSpec((t, H), lambda i: (i, 0)),
        full((H, H)), full((1, H)),
        full((H, H)), full((1, H)),
        full((1, H)), full((1, H)),
        full((H, H2)), full((1, H2)),
        full((H2, NT)), full((1, NT)),
    ]
    args = [tok_chunk] + list(weights)
    aliases = {}
    if carry is not None:
        in_specs += [pl.BlockSpec(memory_space=pl.ANY),
                     pl.BlockSpec(memory_space=pl.ANY)]
        args += [carry[0], carry[1]]
        aliases = {11: 0, 12: 1}

    def body(*refs):
        _mlp_body(*refs[:11], *refs[-6:])

    return pl.pallas_call(
        body,
        grid=(g,),
        in_specs=in_specs,
        scratch_shapes=[
            pltpu.VMEM((t, H), jnp.bfloat16),
            pltpu.VMEM((t, H), jnp.bfloat16),
            pltpu.VMEM((t, H), jnp.bfloat16),
            pltpu.VMEM((t, H2), jnp.bfloat16),
        ],
        out_specs=[
            pl.BlockSpec((t, H), lambda i: (blk_off + i, 0)),
            pl.BlockSpec((t, NT), lambda i: (blk_off + i, 0)),
        ],
        out_shape=[
            jax.ShapeDtypeStruct((n_total, H), jnp.float32),
            jax.ShapeDtypeStruct((n_total, NT), jnp.float32),
        ],
        input_output_aliases=aliases,
        interpret=interpret,
    )(*args)


def kernel(input_ids, table, W1, b1, W2, b2, ln_g, ln_b, Wh1, bh1, Wh2, bh2):
    b, s = input_ids.shape
    n = b * s
    ids = input_ids.reshape(-1).astype(jnp.int32)
    W1b, W2b, Wh1b = _cast_weights(W1, W2, Wh1)
    weights = (W1b, b1.reshape(1, H),
               W2b, b2.reshape(1, H),
               ln_g.reshape(1, H), ln_b.reshape(1, H),
               Wh1b, bh1.reshape(1, H2),
               Wh2.astype(jnp.bfloat16), bh2.reshape(1, NT))
    offs = [0]
    for c in _CHUNKS:
        offs.append(offs[-1] + c)
    toks = [_sc_gather(table, jax.lax.dynamic_slice_in_dim(ids, offs[k], c))
            for k, c in enumerate(_CHUNKS)]
    carry = None
    for k in range(len(_CHUNKS)):
        carry = _tc_mlp_chunk(toks[k], n, offs[k] // _TOK_BLOCK,
                              weights, carry)
    comb, logits = carry
    return comb.reshape(b, s, H), logits.reshape(b, s, NT)


# speedup vs baseline: 1.1153x; 1.0682x over previous
import functools

import jax
import jax.numpy as jnp
from jax.experimental import pallas as pl
from jax.experimental.pallas import tpu as pltpu
from jax.experimental.pallas import tpu_sc as plsc

H = 2048
H2 = 1024
NT = 55

_GATHER_WINDOW = 16
_TOK_BLOCK = 512
_CHUNKS = (4096, 12288)


def _sc_gather(table, ids):
    n = ids.shape[0]
    info = plsc.get_sparse_core_info()
    nw = info.num_cores * info.num_subcores
    per_w = n // nw
    c = _GATHER_WINDOW
    mesh = plsc.VectorSubcoreMesh(core_axis_name="c", subcore_axis_name="s")

    @functools.partial(
        pl.kernel,
        out_type=jax.ShapeDtypeStruct((n, H), jnp.float32),
        mesh=mesh,
        scratch_types=[
            pltpu.VMEM((per_w,), jnp.int32),
            pltpu.VMEM((c, H), jnp.float32),
            pltpu.SemaphoreType.DMA,
        ],
    )
    def gather_kernel(table_hbm, ids_hbm, out_hbm, idx_v, rows_v, sem):
        wid = jax.lax.axis_index("s") * info.num_cores + jax.lax.axis_index("c")
        base = wid * per_w
        pltpu.sync_copy(ids_hbm.at[pl.ds(base, per_w)], idx_v)

        @pl.loop(0, per_w, step=c)
        def _(i):
            pltpu.async_copy(
                table_hbm.at[idx_v.at[pl.ds(i, c)]], rows_v, sem).wait()
            pltpu.sync_copy(rows_v, out_hbm.at[pl.ds(base + i, c)])

    return gather_kernel(table, ids)


def _mlp_body(tok_ref, w1_ref, b1_ref, w2_ref, b2_ref, lng_ref, lnb_ref,
              wh1_ref, bh1_ref, wh2_ref, bh2_ref, comb_ref, log_ref,
              tokb_s, gb_s, lnb_s, rb_s):
    tok = tok_ref[...]
    tokb_s[...] = tok.astype(jnp.bfloat16)
    h1 = jnp.dot(tokb_s[...], w1_ref[...],
                 preferred_element_type=jnp.float32) + b1_ref[...]
    g = 0.5 * h1 * (1.0 + jax.lax.erf(h1 * 0.7071067811865476))
    gb_s[...] = g.astype(jnp.bfloat16)
    h = jnp.dot(gb_s[...], w2_ref[...],
                preferred_element_type=jnp.float32) + b2_ref[...]
    comb = tok + h
    comb_ref[...] = comb
    mu = jnp.mean(comb, axis=-1, keepdims=True)
    var = jnp.mean(comb * comb, axis=-1, keepdims=True) - mu * mu
    ln = (comb - mu) * jax.lax.rsqrt(var + 1e-5) * lng_ref[...] + lnb_ref[...]
    lnb_s[...] = ln.astype(jnp.bfloat16)
    a = jnp.dot(lnb_s[...], wh1_ref[...],
                preferred_element_type=jnp.float32) + bh1_ref[...]
    rb_s[...] = jnp.maximum(a, 0.0).astype(jnp.bfloat16)
    log_ref[...] = jnp.dot(rb_s[...], wh2_ref[...],
                           preferred_element_type=jnp.float32) + bh2_ref[...]


def _cast_weights(W1, W2, Wh1):
    r = 256

    def body(w1_ref, w2_ref, wh1_ref, o1_ref, o2_ref, o3_ref):
        o1_ref[...] = w1_ref[...].astype(jnp.bfloat16)
        o2_ref[...] = w2_ref[...].astype(jnp.bfloat16)
        o3_ref[...] = wh1_ref[...].astype(jnp.bfloat16)

    return pl.pallas_call(
        body,
        grid=(H // r,),
        in_specs=[
            pl.BlockSpec((r, H), lambda i: (i, 0)),
            pl.BlockSpec((r, H), lambda i: (i, 0)),
            pl.BlockSpec((r, H2), lambda i: (i, 0)),
        ],
        out_specs=[
            pl.BlockSpec((r, H), lambda i: (i, 0)),
            pl.BlockSpec((r, H), lambda i: (i, 0)),
            pl.BlockSpec((r, H2), lambda i: (i, 0)),
        ],
        out_shape=[
            jax.ShapeDtypeStruct((H, H), jnp.bfloat16),
            jax.ShapeDtypeStruct((H, H), jnp.bfloat16),
            jax.ShapeDtypeStruct((H, H2), jnp.bfloat16),
        ],
    )(W1, W2, Wh1)


def _tc_mlp_chunk(tok_chunk, n_total, blk_off, weights, carry, interpret=False):
    t = _TOK_BLOCK
    g = tok_chunk.shape[0] // t
    full = lambda shape: pl.BlockSpec(shape, lambda i: (0, 0))
    in_specs = [
        pl.BlockSpec((t, H), lambda i: (i, 0)),
        full((H, H)), full((1, H)),
        full((H, H)), full((1, H)),
        full((1, H)), full((1, H)),
        full((H, H2)), full((1, H2)),
        full((H2, NT)), full((1, NT)),
    ]
    args = [tok_chunk] + list(weights)
    aliases = {}
    if carry is not None:
        in_specs += [pl.BlockSpec(memory_space=pl.ANY),
                     pl.BlockSpec(memory_space=pl.ANY)]
        args += [carry[0], carry[1]]
        aliases = {11: 0, 12: 1}

    def body(*refs):
        _mlp_body(*refs[:11], *refs[-6:])

    return pl.pallas_call(
        body,
        grid=(g,),
        in_specs=in_specs,
        scratch_shapes=[
            pltpu.VMEM((t, H), jnp.bfloat16),
            pltpu.VMEM((t, H), jnp.bfloat16),
            pltpu.VMEM((t, H), jnp.bfloat16),
            pltpu.VMEM((t, H2), jnp.bfloat16),
        ],
        out_specs=[
            pl.BlockSpec((t, H), lambda i: (blk_off + i, 0)),
            pl.BlockSpec((t, NT), lambda i: (blk_off + i, 0)),
        ],
        out_shape=[
            jax.ShapeDtypeStruct((n_total, H), jnp.float32),
            jax.ShapeDtypeStruct((n_total, NT), jnp.float32),
        ],
        input_output_aliases=aliases,
        interpret=interpret,
    )(*args)


def kernel(input_ids, table, W1, b1, W2, b2, ln_g, ln_b, Wh1, bh1, Wh2, bh2):
    b, s = input_ids.shape
    n = b * s
    ids = input_ids.reshape(-1).astype(jnp.int32)
    W1b, W2b, Wh1b = _cast_weights(W1, W2, Wh1)
    weights = (W1b, b1.reshape(1, H),
               W2b, b2.reshape(1, H),
               ln_g.reshape(1, H), ln_b.reshape(1, H),
               Wh1b, bh1.reshape(1, H2),
               Wh2.astype(jnp.bfloat16), bh2.reshape(1, NT))
    offs = [0]
    for c in _CHUNKS:
        offs.append(offs[-1] + c)
    toks = [_sc_gather(table, jax.lax.dynamic_slice_in_dim(ids, offs[k], c))
            for k, c in enumerate(_CHUNKS)]
    carry = None
    for k in range(len(_CHUNKS)):
        carry = _tc_mlp_chunk(toks[k], n, offs[k] // _TOK_BLOCK,
                              weights, carry)
    comb, logits = carry
    return comb.reshape(b, s, H), logits.reshape(b, s, NT)
